# Initial kernel scaffold; baseline (speedup 1.0000x reference)
import jax
import jax.numpy as jnp
from jax.experimental import pallas as pl


def kernel(x, edge_index, edge_type, rule_ids, W_r, W_q_w, W_q_b, W_k_w, W_k_b, rule_emb, bias, ln_gamma, ln_beta):
    raise NotImplementedError("write your pallas kernel here")


# speedup vs baseline: 7.5067x; 7.5067x over previous
import functools

import jax
import jax.numpy as jnp
from jax import lax
from jax.experimental import pallas as pl
from jax.experimental.pallas import tpu as pltpu
from jax.experimental.pallas import tpu_sc as plsc

N = 10000
E = 320000
DIM = 128
NREL = 4

NC = 2
NS = 16
NW = NC * NS
EPT = E // NW
CH = 80
NCHUNK = EPT // CH
NP = 10240
SEG = NP // NS

_SCALE = DIM ** -0.5


def _tc_pre_body(x_ref, wrf_ref, wq_ref, wqb_ref, wkw_ref, wr_ref,
                 hw_ref, qs_ref, k1w_ref):
    xb = x_ref[...]
    hw_ref[...] = jnp.dot(xb, wrf_ref[...], preferred_element_type=jnp.float32)
    q = lax.dot_general(xb, wq_ref[...], (((1,), (1,)), ((), ())),
                        preferred_element_type=jnp.float32)
    qs_ref[...] = (q + wqb_ref[...]) * _SCALE
    wk1 = wkw_ref[:, 0:DIM]
    wk2 = wkw_ref[:, DIM:2 * DIM]
    k1 = lax.dot_general(xb, wk1, (((1,), (1,)), ((), ())),
                         preferred_element_type=jnp.float32)
    rel_mean = jnp.mean(wr_ref[...], axis=-1)
    c2 = lax.dot_general(rel_mean, wk2, (((1,), (1,)), ((), ())),
                         preferred_element_type=jnp.float32)
    k1rep = jnp.concatenate([k1, k1, k1, k1], axis=1)
    k1w_ref[...] = k1rep + c2.reshape(1, NREL * DIM)


def _tc_precompute(x, wrf, wq, wqb, wkw, wr):
    B = 1000
    grid = (N // B,)
    return pl.pallas_call(
        _tc_pre_body,
        grid=grid,
        in_specs=[
            pl.BlockSpec((B, DIM), lambda i: (i, 0)),
            pl.BlockSpec((DIM, NREL * DIM), lambda i: (0, 0)),
            pl.BlockSpec((DIM, DIM), lambda i: (0, 0)),
            pl.BlockSpec((1, DIM), lambda i: (0, 0)),
            pl.BlockSpec((DIM, 3 * DIM), lambda i: (0, 0)),
            pl.BlockSpec((NREL, DIM, DIM), lambda i: (0, 0, 0)),
        ],
        out_specs=[
            pl.BlockSpec((B, NREL * DIM), lambda i: (i, 0)),
            pl.BlockSpec((B, DIM), lambda i: (i, 0)),
            pl.BlockSpec((B, NREL * DIM), lambda i: (i, 0)),
        ],
        out_shape=[
            jax.ShapeDtypeStruct((N, NREL * DIM), jnp.float32),
            jax.ShapeDtypeStruct((N, DIM), jnp.float32),
            jax.ShapeDtypeStruct((N, NREL * DIM), jnp.float32),
        ],
    )(x, wrf, wq, wqb, wkw, wr)


_MESH = plsc.VectorSubcoreMesh(core_axis_name="c", subcore_axis_name="s")


@functools.partial(
    pl.kernel,
    out_type=[
        jax.ShapeDtypeStruct((E,), jnp.float32),
        jax.ShapeDtypeStruct((NW, 16), jnp.float32),
    ],
    mesh=_MESH,
    compiler_params=pltpu.CompilerParams(needs_layout_passes=False),
    scratch_types=[
        pltpu.VMEM((CH,), jnp.int32),
        pltpu.VMEM((CH,), jnp.int32),
        pltpu.VMEM((CH,), jnp.int32),
        pltpu.VMEM((CH,), jnp.int32),
        pltpu.VMEM((CH, DIM), jnp.float32),
        pltpu.VMEM((CH, DIM), jnp.float32),
        pltpu.VMEM((CH,), jnp.float32),
        pltpu.VMEM((16,), jnp.float32),
        pltpu.SemaphoreType.DMA,
        pltpu.SemaphoreType.DMA,
    ],
)
def _sc_pass1(qs_hbm, k1f_hbm, src_hbm, dst_hbm, et_hbm,
              base_hbm, maxt_hbm,
              srcb, dstb, etb, kidxb, krows, qrows, baseb, maxbuf,
              sem1, sem2):
    wid = lax.axis_index("s") * NC + lax.axis_index("c")
    ebase = wid * EPT
    lanes = lax.broadcasted_iota(jnp.int32, (16,), 0)

    def chunk(j, maxacc):
        off = ebase + j * CH
        pltpu.sync_copy(src_hbm.at[pl.ds(off, CH)], srcb)
        pltpu.sync_copy(dst_hbm.at[pl.ds(off, CH)], dstb)
        pltpu.sync_copy(et_hbm.at[pl.ds(off, CH)], etb)
        for g in range(CH // 16):
            sl = pl.ds(g * 16, 16)
            kidxb[sl] = srcb[sl] * NREL + etb[sl]
        pltpu.async_copy(k1f_hbm.at[kidxb], krows, sem1).wait()
        pltpu.async_copy(qs_hbm.at[dstb], qrows, sem2).wait()
        for g in range(CH // 16):
            rowix = g * 16 + lanes

            def dbody(d, acc):
                col = lax.broadcast(d, (16,))
                kv = plsc.load_gather(krows, [rowix, col])
                qv = plsc.load_gather(qrows, [rowix, col])
                return acc + kv * qv

            acc = lax.fori_loop(0, DIM, dbody, jnp.zeros((16,), jnp.float32),
                                unroll=8)
            baseb[pl.ds(g * 16, 16)] = acc
            maxacc = jnp.maximum(maxacc, acc)
        pltpu.sync_copy(baseb, base_hbm.at[pl.ds(off, CH)])
        return maxacc

    init = jnp.full((16,), -jnp.inf, jnp.float32)
    maxacc = lax.fori_loop(0, NCHUNK, chunk, init)
    maxbuf[...] = lax.broadcast(jnp.max(maxacc), (16,))
    pltpu.sync_copy(maxbuf, maxt_hbm.at[wid])


@functools.partial(
    pl.kernel,
    out_type=[
        jax.ShapeDtypeStruct((E,), jnp.float32),
        jax.ShapeDtypeStruct((NC, NP), jnp.float32),
    ],
    mesh=_MESH,
    compiler_params=pltpu.CompilerParams(needs_layout_passes=False),
    scratch_types=[
        pltpu.VMEM((CH,), jnp.float32),
        pltpu.VMEM((CH,), jnp.float32),
        pltpu.VMEM((CH,), jnp.int32),
        pltpu.VMEM((NW, 16), jnp.float32),
        pltpu.VMEM((SEG,), jnp.float32),
        pltpu.VMEM_SHARED((NP,), jnp.float32),
    ],
)
def _sc_pass2(base_hbm, dst_hbm, maxt_hbm,
              ex_hbm, den2_hbm,
              basev, exb, dstb, mall, zbuf, den_sp):
    c = lax.axis_index("c")
    s = lax.axis_index("s")
    wid = s * NC + c
    ebase = wid * EPT

    pltpu.sync_copy(maxt_hbm, mall)
    mv = mall[0]
    for i in range(1, NW):
        mv = jnp.maximum(mv, mall[i])
    gmax = jnp.max(mv)

    for i in range(SEG // 16):
        zbuf[pl.ds(i * 16, 16)] = jnp.zeros((16,), jnp.float32)
    pltpu.sync_copy(zbuf, den_sp.at[pl.ds(s * SEG, SEG)])
    plsc.subcore_barrier()

    def chunk(j, carry):
        off = ebase + j * CH
        pltpu.sync_copy(base_hbm.at[pl.ds(off, CH)], basev)
        pltpu.sync_copy(dst_hbm.at[pl.ds(off, CH)], dstb)
        for g in range(CH // 16):
            sl = pl.ds(g * 16, 16)
            exb[sl] = jnp.exp(basev[sl] - gmax)
        pltpu.sync_copy(exb, ex_hbm.at[pl.ds(off, CH)])
        pltpu.sync_copy(exb, den_sp.at[dstb], add=True)
        return carry

    lax.fori_loop(0, NCHUNK, chunk, 0)
    plsc.subcore_barrier()

    pltpu.sync_copy(den_sp.at[pl.ds(s * SEG, SEG)], zbuf)
    pltpu.sync_copy(zbuf, den2_hbm.at[c, pl.ds(s * SEG, SEG)])


@functools.partial(
    pl.kernel,
    out_type=jax.ShapeDtypeStruct((NC, NP, DIM), jnp.float32),
    mesh=_MESH,
    compiler_params=pltpu.CompilerParams(needs_layout_passes=False),
    scratch_types=[
        pltpu.VMEM((CH,), jnp.int32),
        pltpu.VMEM((CH,), jnp.int32),
        pltpu.VMEM((CH,), jnp.int32),
        pltpu.VMEM((CH,), jnp.int32),
        pltpu.VMEM((CH,), jnp.float32),
        pltpu.VMEM((CH + 16,), jnp.float32),
        pltpu.VMEM((CH, DIM), jnp.float32),
        pltpu.VMEM((CH, DIM), jnp.float32),
        pltpu.VMEM((NP,), jnp.float32),
        pltpu.VMEM((NP,), jnp.float32),
        pltpu.VMEM_SHARED((NP, DIM), jnp.float32),
        pltpu.SemaphoreType.DMA,
    ],
)
def _sc_pass3(ex_hbm, src_hbm, dst_hbm, et_hbm, den2_hbm, hf_hbm,
              outp_hbm,
              srcb, dstb, etb, kidxb, exv, attnb, hrows, zrows, dloc, dtmp,
              out_sp, sem1):
    c = lax.axis_index("c")
    s = lax.axis_index("s")
    wid = s * NC + c
    ebase = wid * EPT

    pltpu.sync_copy(den2_hbm.at[0], dloc)
    pltpu.sync_copy(den2_hbm.at[1], dtmp)

    def addden(i, carry):
        sl = pl.ds(i * 16, 16)
        dloc[sl] = dloc[sl] + dtmp[sl]
        return carry

    lax.fori_loop(0, NP // 16, addden, 0)

    def zrow(i, carry):
        for g in range(DIM // 16):
            zrows[i, pl.ds(g * 16, 16)] = jnp.zeros((16,), jnp.float32)
        return carry

    lax.fori_loop(0, CH, zrow, 0)
    for k in range(SEG // CH):
        pltpu.sync_copy(zrows, out_sp.at[pl.ds(s * SEG + k * CH, CH)])
    plsc.subcore_barrier()

    def chunk(j, carry):
        off = ebase + j * CH
        pltpu.sync_copy(ex_hbm.at[pl.ds(off, CH)], exv)
        pltpu.sync_copy(src_hbm.at[pl.ds(off, CH)], srcb)
        pltpu.sync_copy(dst_hbm.at[pl.ds(off, CH)], dstb)
        pltpu.sync_copy(et_hbm.at[pl.ds(off, CH)], etb)
        for g in range(CH // 16):
            sl = pl.ds(g * 16, 16)
            kidxb[sl] = srcb[sl] * NREL + etb[sl]
        pltpu.async_copy(hf_hbm.at[kidxb], hrows, sem1).wait()
        for g in range(CH // 16):
            sl = pl.ds(g * 16, 16)
            dv = plsc.load_gather(dloc, [dstb[sl]])
            attnb[sl] = exv[sl] / (dv + 1e-16)

        def scale(e, carry2):
            awin = attnb[pl.ds(e, 16)]
            av = lax.broadcast(awin[0], (16,))
            for g in range(DIM // 16):
                sl = pl.ds(g * 16, 16)
                hrows[e, sl] = hrows[e, sl] * av
            return carry2

        lax.fori_loop(0, CH, scale, 0)
        pltpu.sync_copy(hrows, out_sp.at[dstb], add=True)
        return carry

    lax.fori_loop(0, NCHUNK, chunk, 0)
    plsc.subcore_barrier()

    for k in range(SEG // CH):
        base_row = s * SEG + k * CH
        pltpu.sync_copy(out_sp.at[pl.ds(base_row, CH)], hrows)
        pltpu.sync_copy(hrows, outp_hbm.at[c, pl.ds(base_row, CH)])


def _tc_epi_body(op_ref, b_ref, g_ref, bt_ref, o_ref):
    sm = op_ref[0] + op_ref[1] + b_ref[...]
    mu = jnp.mean(sm, axis=-1, keepdims=True)
    var = jnp.mean((sm - mu) ** 2, axis=-1, keepdims=True)
    y = (sm - mu) / jnp.sqrt(var + 1e-5) * g_ref[...] + bt_ref[...]
    o_ref[...] = jnp.maximum(y, 0.0)


def _tc_epilogue(outp, bias, gamma, beta):
    B = 1000
    grid = (N // B,)
    return pl.pallas_call(
        _tc_epi_body,
        grid=grid,
        in_specs=[
            pl.BlockSpec((NC, B, DIM), lambda i: (0, i, 0)),
            pl.BlockSpec((1, DIM), lambda i: (0, 0)),
            pl.BlockSpec((1, DIM), lambda i: (0, 0)),
            pl.BlockSpec((1, DIM), lambda i: (0, 0)),
        ],
        out_specs=pl.BlockSpec((B, DIM), lambda i: (i, 0)),
        out_shape=jax.ShapeDtypeStruct((N, DIM), jnp.float32),
    )(outp, bias, gamma, beta)


def kernel(x, edge_index, edge_type, rule_ids, W_r, W_q_w, W_q_b, W_k_w,
           W_k_b, rule_emb, bias, ln_gamma, ln_beta):
    src = edge_index[0]
    dst = edge_index[1]
    wrf = W_r.transpose(1, 0, 2).reshape(DIM, NREL * DIM)
    hw, qs, k1w = _tc_precompute(x, wrf, W_q_w, W_q_b.reshape(1, DIM),
                                 W_k_w, W_r)
    hf = hw.reshape(N * NREL, DIM)
    k1f = k1w.reshape(N * NREL, DIM)
    base, maxt = _sc_pass1(qs, k1f, src, dst, edge_type)
    ex, den2 = _sc_pass2(base, dst, maxt)
    outp = _sc_pass3(ex, src, dst, edge_type, den2, hf)
    return _tc_epilogue(outp, bias.reshape(1, DIM), ln_gamma.reshape(1, DIM),
                        ln_beta.reshape(1, DIM))
